# baseline (speedup 1.0000x reference)
import jax
import jax.numpy as jnp
from jax import lax
from jax.experimental import pallas as pl
from jax.experimental.pallas import tpu as pltpu

_D_MODEL = 4096
_N_EXP = 64
_K = 8
_BLOCK = 512
_SPLIT = 2
_SUB = _BLOCK // _SPLIT


def _topk_gates(rw):
    iota = lax.broadcasted_iota(jnp.int32, rw.shape, 1).astype(jnp.float32)
    work = rw
    v0 = None
    big = jnp.float32(2.0 * _N_EXP)
    neg_inf = jnp.float32(-jnp.inf)
    for t in range(_K):
        mx = jnp.max(work, axis=1, keepdims=True)
        if t == 0:
            v0 = mx
        masked = jnp.where(work == mx, iota, big)
        idx = jnp.min(masked, axis=1, keepdims=True)
        work = jnp.where(masked == idx, neg_inf, work)

    e_full = jnp.where(work == neg_inf, jnp.exp(rw - v0), 0.0)
    denom = jnp.sum(e_full, axis=1, keepdims=True)
    return e_full * (1.0 / denom)


def _router_kernel(*refs):
    x_refs = refs[:_SPLIT]
    w_ref, b_ref, gates_ref, rw_ref = refs[_SPLIT:]
    w = w_ref[...]
    bias = b_ref[...]
    for s in range(_SPLIT):
        rw = lax.dot_general(
            x_refs[s][...], w, (((1,), (1,)), ((), ())),
            preferred_element_type=jnp.float32,
        ) + bias
        rw_ref[pl.ds(s * _SUB, _SUB), :] = rw
        gates_ref[pl.ds(s * _SUB, _SUB), :] = _topk_gates(rw)


def kernel(x, W, b):
    n_tokens = x.shape[0]
    grid = (n_tokens // _BLOCK,)
    b2 = b.reshape(1, _N_EXP)

    def _x_spec(s):
        return pl.BlockSpec(
            (_SUB, _D_MODEL), lambda i, s=s: (_SPLIT * i + s, 0)
        )

    gates, rw = pl.pallas_call(
        _router_kernel,
        grid=grid,
        in_specs=[_x_spec(s) for s in range(_SPLIT)] + [
            pl.BlockSpec((_N_EXP, _D_MODEL), lambda i: (0, 0)),
            pl.BlockSpec((1, _N_EXP), lambda i: (0, 0)),
        ],
        out_specs=[
            pl.BlockSpec((_BLOCK, _N_EXP), lambda i: (i, 0)),
            pl.BlockSpec((_BLOCK, _N_EXP), lambda i: (i, 0)),
        ],
        out_shape=[
            jax.ShapeDtypeStruct((n_tokens, _N_EXP), jnp.float32),
            jax.ShapeDtypeStruct((n_tokens, _N_EXP), jnp.float32),
        ],
        compiler_params=pltpu.CompilerParams(
            dimension_semantics=("parallel",),
        ),
    )(*([x] * _SPLIT + [W, b2]))
    return (gates, rw)


# speedup vs baseline: 1.1483x; 1.1483x over previous
import jax
import jax.numpy as jnp
from jax import lax
from jax.experimental import pallas as pl
from jax.experimental.pallas import tpu as pltpu

_D_MODEL = 4096
_N_EXP = 64
_K = 8
_BLOCK = 1024
_SPLIT = 4
_SUB = _BLOCK // _SPLIT


def _topk_gates(rw):
    iota = lax.broadcasted_iota(jnp.int32, rw.shape, 1).astype(jnp.float32)
    work = rw
    v0 = None
    big = jnp.float32(2.0 * _N_EXP)
    neg_inf = jnp.float32(-jnp.inf)
    for t in range(_K):
        mx = jnp.max(work, axis=1, keepdims=True)
        if t == 0:
            v0 = mx
        masked = jnp.where(work == mx, iota, big)
        idx = jnp.min(masked, axis=1, keepdims=True)
        work = jnp.where(masked == idx, neg_inf, work)

    e_full = jnp.where(work == neg_inf, jnp.exp(rw - v0), 0.0)
    denom = jnp.sum(e_full, axis=1, keepdims=True)
    return e_full * (1.0 / denom)


def _router_kernel(*refs):
    x_refs = refs[:_SPLIT]
    w_ref, b_ref, gates_ref, rw_ref = refs[_SPLIT:]
    w = w_ref[...]
    bias = b_ref[...]
    for s in range(_SPLIT):
        rw = lax.dot_general(
            x_refs[s][...], w, (((1,), (1,)), ((), ())),
            preferred_element_type=jnp.float32,
        ) + bias
        rw_ref[pl.ds(s * _SUB, _SUB), :] = rw
        gates_ref[pl.ds(s * _SUB, _SUB), :] = _topk_gates(rw)


def kernel(x, W, b):
    n_tokens = x.shape[0]
    grid = (n_tokens // _BLOCK,)
    b2 = b.reshape(1, _N_EXP)

    def _x_spec(s):
        return pl.BlockSpec(
            (_SUB, _D_MODEL), lambda i, s=s: (_SPLIT * i + s, 0)
        )

    gates, rw = pl.pallas_call(
        _router_kernel,
        grid=grid,
        in_specs=[_x_spec(s) for s in range(_SPLIT)] + [
            pl.BlockSpec((_N_EXP, _D_MODEL), lambda i: (0, 0)),
            pl.BlockSpec((1, _N_EXP), lambda i: (0, 0)),
        ],
        out_specs=[
            pl.BlockSpec((_BLOCK, _N_EXP), lambda i: (i, 0)),
            pl.BlockSpec((_BLOCK, _N_EXP), lambda i: (i, 0)),
        ],
        out_shape=[
            jax.ShapeDtypeStruct((n_tokens, _N_EXP), jnp.float32),
            jax.ShapeDtypeStruct((n_tokens, _N_EXP), jnp.float32),
        ],
        compiler_params=pltpu.CompilerParams(
            dimension_semantics=("parallel",),
        ),
    )(*([x] * _SPLIT + [W, b2]))
    return (gates, rw)


# speedup vs baseline: 1.1494x; 1.0009x over previous
import jax
import jax.numpy as jnp
from jax import lax
from jax.experimental import pallas as pl
from jax.experimental.pallas import tpu as pltpu

_D_MODEL = 4096
_N_EXP = 64
_K = 8
_BLOCK = 1024
_SPLIT = 4
_SUB = _BLOCK // _SPLIT


def _topk_gates(rw):
    iota = lax.broadcasted_iota(jnp.int32, rw.shape, 1).astype(jnp.float32)
    work = rw
    v0 = None
    big = jnp.float32(2.0 * _N_EXP)
    neg_inf = jnp.float32(-jnp.inf)
    for t in range(_K):
        mx = jnp.max(work, axis=1, keepdims=True)
        if t == 0:
            v0 = mx
        masked = jnp.where(work == mx, iota, big)
        idx = jnp.min(masked, axis=1, keepdims=True)
        work = jnp.where(masked == idx, neg_inf, work)

    e_full = jnp.where(work == neg_inf, jnp.exp(rw - v0), 0.0)
    denom = jnp.sum(e_full, axis=1, keepdims=True)
    return e_full * (1.0 / denom)


def _router_kernel(*refs):
    x_refs = refs[:_SPLIT]
    w_ref, b_ref, gates_ref, rw_ref = refs[_SPLIT:]
    w = w_ref[...]
    bias = b_ref[...]
    for s in range(_SPLIT):
        rw = lax.dot_general(
            x_refs[s][...], w, (((1,), (1,)), ((), ())),
            preferred_element_type=jnp.float32,
        ) + bias
        rw_ref[pl.ds(s * _SUB, _SUB), :] = rw
        gates_ref[pl.ds(s * _SUB, _SUB), :] = _topk_gates(rw)


def kernel(x, W, b):
    n_tokens = x.shape[0]
    grid = (n_tokens // _BLOCK,)
    b2 = b.reshape(1, _N_EXP)

    def _x_spec(s):
        return pl.BlockSpec(
            (_SUB, _D_MODEL), lambda i, s=s: (_SPLIT * i + s, 0)
        )

    gates, rw = pl.pallas_call(
        _router_kernel,
        grid=grid,
        in_specs=[_x_spec(s) for s in range(_SPLIT)] + [
            pl.BlockSpec((_N_EXP, _D_MODEL), lambda i: (0, 0)),
            pl.BlockSpec((1, _N_EXP), lambda i: (0, 0)),
        ],
        out_specs=[
            pl.BlockSpec((_BLOCK, _N_EXP), lambda i: (i, 0)),
            pl.BlockSpec((_BLOCK, _N_EXP), lambda i: (i, 0)),
        ],
        out_shape=[
            jax.ShapeDtypeStruct((n_tokens, _N_EXP), jnp.float32),
            jax.ShapeDtypeStruct((n_tokens, _N_EXP), jnp.float32),
        ],
        compiler_params=pltpu.CompilerParams(
            dimension_semantics=("arbitrary",),
        ),
    )(*([x] * _SPLIT + [W, b2]))
    return (gates, rw)
